# baseline (speedup 1.0000x reference)
import functools

import jax
import jax.numpy as jnp
from jax.experimental import pallas as pl
from jax.experimental.pallas import tpu as pltpu
from jax.experimental.pallas import tpu_sc as plsc

N = 16384
C = 1000
K = N // 2

N_TC = 8192
N_SC = N - N_TC
BLOCK_R = 2048
NUM_B = N_TC // BLOCK_R

NW = 32
RW = N_SC // NW
GROUPS = RW // 16
UNROLL = 8


def _tc_losses_kernel(x_ref, lab_ref, out_ref):
    x = x_ref[...]
    m = jnp.max(x, axis=1, keepdims=True)
    se = jnp.sum(jnp.exp(x - m), axis=1)
    lbl = lab_ref[0, 0, :]
    cols = jax.lax.broadcasted_iota(jnp.int32, (BLOCK_R, C), 1)
    xlab = jnp.sum(jnp.where(cols == lbl[:, None], x, 0.0), axis=1)
    out_ref[0, 0, :] = jnp.log(se) + m[:, 0] - xlab


def _tc_losses(logits, labels32):
    labels3 = labels32[:N_TC].reshape(NUM_B, 1, BLOCK_R)
    return pl.pallas_call(
        _tc_losses_kernel,
        grid=(NUM_B,),
        in_specs=[
            pl.BlockSpec((BLOCK_R, C), lambda i: (i, 0)),
            pl.BlockSpec((1, 1, BLOCK_R), lambda i: (i, 0, 0)),
        ],
        out_specs=pl.BlockSpec((1, 1, BLOCK_R), lambda i: (i, 0, 0)),
        out_shape=jax.ShapeDtypeStruct((NUM_B, 1, BLOCK_R), jnp.float32),
    )(logits, labels3)


_sc_mesh = plsc.VectorSubcoreMesh(core_axis_name="c", subcore_axis_name="s")


@functools.partial(
    pl.kernel,
    mesh=_sc_mesh,
    out_type=(
        jax.ShapeDtypeStruct((N_SC,), jnp.float32),
        jax.ShapeDtypeStruct((N_SC,), jnp.float32),
    ),
    scratch_types=[
        pltpu.VMEM((16 * C,), jnp.float32),
        pltpu.VMEM((16 * C,), jnp.float32),
        pltpu.VMEM((RW,), jnp.int32),
        pltpu.VMEM((RW,), jnp.float32),
        pltpu.VMEM((RW,), jnp.float32),
        pltpu.SemaphoreType.DMA,
        pltpu.SemaphoreType.DMA,
    ],
    compiler_params=pltpu.CompilerParams(
        use_tc_tiling_on_sc=False, needs_layout_passes=False),
)
def _sc_losses(logits_hbm, labels_hbm, a_hbm, b_hbm,
               buf0, buf1, lbl_v, a_v, b_v, sem0, sem1):
    c = jax.lax.axis_index("c")
    s = jax.lax.axis_index("s")
    wid = s * 2 + c
    base = wid * RW
    row0 = N_TC + base
    pltpu.sync_copy(labels_hbm.at[pl.ds(row0, RW)], lbl_v)

    bufs = (buf0, buf1)
    sems = (sem0, sem1)
    rowbase = jax.lax.broadcasted_iota(jnp.int32, (16,), 0) * C

    def start_group(g):
        return [
            pltpu.async_copy(
                logits_hbm.at[row0 + g * 16 + r],
                bufs[g % 2].at[pl.ds(r * C, C)],
                sems[g % 2])
            for r in range(16)
        ]

    copies = [None] * GROUPS
    copies[0] = start_group(0)
    for g in range(GROUPS):
        if g + 1 < GROUPS:
            copies[g + 1] = start_group(g + 1)
        for cp in copies[g]:
            cp.wait()
        buf = bufs[g % 2]

        def body1(j, macc):
            for u in range(UNROLL):
                idx = rowbase + j * UNROLL + u
                v = plsc.load_gather(buf, [idx])
                macc = jnp.maximum(macc, v)
            return macc

        macc = jax.lax.fori_loop(
            0, C // UNROLL, body1, jnp.full((16,), -3.0e38, jnp.float32))

        def body2(j, sacc):
            for u in range(UNROLL):
                idx = rowbase + j * UNROLL + u
                v = plsc.load_gather(buf, [idx])
                sacc = sacc + jnp.exp(v - macc)
            return sacc

        sacc = jax.lax.fori_loop(
            0, C // UNROLL, body2, jnp.zeros((16,), jnp.float32))

        lbl = lbl_v[pl.ds(g * 16, 16)]
        xv = plsc.load_gather(buf, [rowbase + lbl])
        a_v[pl.ds(g * 16, 16)] = macc - xv
        b_v[pl.ds(g * 16, 16)] = sacc

    pltpu.sync_copy(a_v, a_hbm.at[pl.ds(base, RW)])
    pltpu.sync_copy(b_v, b_hbm.at[pl.ds(base, RW)])


def _finalize_kernel(lt_ref, a_ref, b_ref, out_ref):
    lt = lt_ref[...]
    lsc = jnp.log(b_ref[...]) + a_ref[...]
    bits1 = jax.lax.bitcast_convert_type(lt, jnp.int32)
    bits2 = jax.lax.bitcast_convert_type(lsc, jnp.int32)

    def body(_, carry):
        lo, hi = carry
        mid = lo + (hi - lo) // 2
        cnt = (jnp.sum((bits1 >= mid).astype(jnp.int32)) +
               jnp.sum((bits2 >= mid).astype(jnp.int32)))
        take = cnt >= K
        return jnp.where(take, mid, lo), jnp.where(take, hi, mid)

    t_int, _ = jax.lax.fori_loop(
        0, 31, body, (jnp.int32(0), jnp.int32(0x7F800000)))
    t = jax.lax.bitcast_convert_type(t_int, jnp.float32)
    gt1 = bits1 > t_int
    gt2 = bits2 > t_int
    cnt_gt = (jnp.sum(gt1.astype(jnp.int32)) +
              jnp.sum(gt2.astype(jnp.int32)))
    sum_gt = (jnp.sum(jnp.where(gt1, lt, 0.0)) +
              jnp.sum(jnp.where(gt2, lsc, 0.0)))
    out_ref[0, 0] = (sum_gt + (K - cnt_gt).astype(jnp.float32) * t) / K


def _finalize(lt, a, b):
    return pl.pallas_call(
        _finalize_kernel,
        out_specs=pl.BlockSpec(memory_space=pltpu.SMEM),
        out_shape=jax.ShapeDtypeStruct((1, 1), jnp.float32),
    )(lt.reshape(N_TC // 128, 128),
      a.reshape(N_SC // 128, 128),
      b.reshape(N_SC // 128, 128))


@jax.jit
def kernel(logits, labels):
    labels32 = labels.astype(jnp.int32)
    a, b = _sc_losses(logits, labels32)
    lt = _tc_losses(logits, labels32)
    return _finalize(lt, a, b)[0, 0]


# speedup vs baseline: 2.2154x; 2.2154x over previous
import functools

import jax
import jax.numpy as jnp
from jax.experimental import pallas as pl
from jax.experimental.pallas import tpu as pltpu

N = 16384
C = 1000
K = N // 2
BLOCK_R = 2048
NUM_B = N // BLOCK_R


def _losses_kernel(x_ref, lab_ref, out_ref):
    x = x_ref[...]
    m = jnp.max(x, axis=1, keepdims=True)
    se = jnp.sum(jnp.exp(x - m), axis=1)
    lbl = lab_ref[0, 0, :]
    cols = jax.lax.broadcasted_iota(jnp.int32, (BLOCK_R, C), 1)
    xlab = jnp.sum(jnp.where(cols == lbl[:, None], x, 0.0), axis=1)
    out_ref[0, 0, :] = jnp.log(se) + m[:, 0] - xlab


def _losses(logits, labels32):
    labels3 = labels32.reshape(NUM_B, 1, BLOCK_R)
    return pl.pallas_call(
        _losses_kernel,
        grid=(NUM_B,),
        in_specs=[
            pl.BlockSpec((BLOCK_R, C), lambda i: (i, 0)),
            pl.BlockSpec((1, 1, BLOCK_R), lambda i: (i, 0, 0)),
        ],
        out_specs=pl.BlockSpec((1, 1, BLOCK_R), lambda i: (i, 0, 0)),
        out_shape=jax.ShapeDtypeStruct((NUM_B, 1, BLOCK_R), jnp.float32),
    )(logits, labels3)


def _finalize_kernel(l_ref, out_ref):
    vals = l_ref[...]
    bits = jax.lax.bitcast_convert_type(vals, jnp.int32)

    def body(_, carry):
        lo, hi = carry
        mid = lo + (hi - lo) // 2
        cnt = jnp.sum((bits >= mid).astype(jnp.int32))
        take = cnt >= K
        return jnp.where(take, mid, lo), jnp.where(take, hi, mid)

    t_int, _ = jax.lax.fori_loop(
        0, 31, body, (jnp.int32(0), jnp.int32(0x7F800000)))
    t = jax.lax.bitcast_convert_type(t_int, jnp.float32)
    gt = bits > t_int
    cnt_gt = jnp.sum(gt.astype(jnp.int32))
    sum_gt = jnp.sum(jnp.where(gt, vals, 0.0))
    out_ref[0, 0] = (sum_gt + (K - cnt_gt).astype(jnp.float32) * t) / K


def _finalize(losses):
    return pl.pallas_call(
        _finalize_kernel,
        out_specs=pl.BlockSpec(memory_space=pltpu.SMEM),
        out_shape=jax.ShapeDtypeStruct((1, 1), jnp.float32),
    )(losses.reshape(N // 128, 128))


@jax.jit
def kernel(logits, labels):
    labels32 = labels.astype(jnp.int32)
    losses = _losses(logits, labels32)
    return _finalize(losses)[0, 0]
